# baseline (speedup 1.0000x reference)
import functools

import jax
import jax.numpy as jnp
from jax import lax
from jax.experimental import pallas as pl
from jax.experimental.pallas import tpu as pltpu
from jax.experimental.pallas import tpu_sc as plsc

VOCAB = 30522
HIDDEN = 768
MAX_POS = 512
EPS = 1e-12
L = 16
NF = HIDDEN // L
CHUNK = 64


def _rsqrt16(a):
    i = lax.bitcast_convert_type(a, jnp.int32)
    y = lax.bitcast_convert_type(jnp.int32(0x5F3759DF) - (i >> 1), jnp.float32)
    for _ in range(3):
        y = y * (1.5 - 0.5 * a * y * y)
    return y


def _make_sc_kernel(n_tokens, n_workers):
    tok_per_w = n_tokens // n_workers
    n_chunks = tok_per_w // CHUNK
    mesh = plsc.VectorSubcoreMesh(core_axis_name="c", subcore_axis_name="s")

    @functools.partial(
        pl.kernel,
        mesh=mesh,
        out_type=jax.ShapeDtypeStruct((n_tokens, HIDDEN), jnp.float32),
        scratch_types=[
            pltpu.VMEM((CHUNK,), jnp.int32),
            pltpu.VMEM((CHUNK, HIDDEN), jnp.float32),
            pltpu.VMEM((CHUNK, HIDDEN), jnp.float32),
            pltpu.VMEM((HIDDEN,), jnp.float32),
            pltpu.VMEM((HIDDEN,), jnp.float32),
            pltpu.SemaphoreType.DMA,
        ],
        compiler_params=pltpu.CompilerParams(needs_layout_passes=False),
    )
    def body(ids_hbm, table_hbm, pos_hbm, gamma_hbm, beta_hbm, out_hbm,
             idx_v, rows_v, pos_v, gamma_v, beta_v, sem):
        nc = 2
        wid = lax.axis_index("s") * nc + lax.axis_index("c")
        pltpu.sync_copy(gamma_hbm, gamma_v)
        pltpu.sync_copy(beta_hbm, beta_v)

        def chunk_body(c, carry):
            base = pl.multiple_of(wid * tok_per_w + c * CHUNK, CHUNK)
            pbase = pl.multiple_of(c * CHUNK, CHUNK)
            pltpu.sync_copy(ids_hbm.at[pl.ds(base, CHUNK)], idx_v)
            pltpu.async_copy(table_hbm.at[idx_v], rows_v, sem).wait()
            pltpu.sync_copy(pos_hbm.at[pl.ds(pbase, CHUNK)], pos_v)

            def tok_body(t, tcarry):
                s = jnp.zeros((L,), jnp.float32)
                s2 = jnp.zeros((L,), jnp.float32)
                for i in range(NF):
                    sl = pl.ds(i * L, L)
                    v = rows_v[t, sl] + pos_v[t, sl]
                    rows_v[t, sl] = v
                    s = s + v
                    s2 = s2 + v * v
                tot = jnp.sum(s)
                tot2 = jnp.sum(s2)
                mean = tot * (1.0 / HIDDEN)
                var = tot2 * (1.0 / HIDDEN) - mean * mean
                mean_v = jnp.full((L,), mean, jnp.float32)
                rstd_v = _rsqrt16(jnp.full((L,), var + EPS, jnp.float32))
                for i in range(NF):
                    sl = pl.ds(i * L, L)
                    x = rows_v[t, sl]
                    rows_v[t, sl] = ((x - mean_v) * rstd_v * gamma_v[sl]
                                     + beta_v[sl])
                return tcarry

            lax.fori_loop(0, CHUNK, tok_body, 0)
            pltpu.sync_copy(rows_v, out_hbm.at[pl.ds(base, CHUNK)])
            return carry

        lax.fori_loop(0, n_chunks, chunk_body, 0)

    return body


def kernel(input_ids, word_emb, pos_emb, ln_gamma, ln_beta):
    b, s = input_ids.shape
    n_tokens = b * s
    info = plsc.get_sparse_core_info()
    n_workers = info.num_cores * info.num_subcores
    ids = input_ids.reshape(n_tokens).astype(jnp.int32)
    sc = _make_sc_kernel(n_tokens, n_workers)
    out = sc(ids, word_emb, pos_emb, ln_gamma, ln_beta)
    return out.reshape(b, s, HIDDEN)


# speedup vs baseline: 2.1775x; 2.1775x over previous
import functools

import jax
import jax.numpy as jnp
from jax import lax
from jax.experimental import pallas as pl
from jax.experimental.pallas import tpu as pltpu
from jax.experimental.pallas import tpu_sc as plsc

HIDDEN = 768
EPS = 1e-12
L = 16
NF = HIDDEN // L
CHUNK = 32
TUNROLL = 4


def _rsqrt_scalar(a):
    i = lax.bitcast_convert_type(a, jnp.int32)
    y = lax.bitcast_convert_type(jnp.int32(0x5F3759DF) - (i >> 1), jnp.float32)
    for _ in range(3):
        y = y * (1.5 - 0.5 * a * y * y)
    return y


def _make_sc_kernel(n_tokens, n_workers, seq_len):
    tok_per_w = n_tokens // n_workers
    n_chunks = tok_per_w // CHUNK
    mesh = plsc.VectorSubcoreMesh(core_axis_name="c", subcore_axis_name="s")

    @functools.partial(
        pl.kernel,
        mesh=mesh,
        out_type=jax.ShapeDtypeStruct((n_tokens, HIDDEN), jnp.float32),
        scratch_types=[
            pltpu.VMEM((tok_per_w,), jnp.int32),
            pltpu.VMEM((CHUNK, HIDDEN), jnp.float32),
            pltpu.VMEM((CHUNK, HIDDEN), jnp.float32),
            pltpu.VMEM((CHUNK, HIDDEN), jnp.float32),
            pltpu.VMEM((CHUNK, HIDDEN), jnp.float32),
            pltpu.VMEM((HIDDEN,), jnp.float32),
            pltpu.VMEM((HIDDEN,), jnp.float32),
            pltpu.VMEM_SHARED((seq_len, HIDDEN), jnp.float32),
            pltpu.SMEM((CHUNK,), jnp.float32),
            pltpu.SMEM((CHUNK,), jnp.float32),
            pltpu.SemaphoreType.DMA,
            pltpu.SemaphoreType.DMA,
            pltpu.SemaphoreType.DMA,
            pltpu.SemaphoreType.DMA,
            pltpu.SemaphoreType.DMA,
            pltpu.SemaphoreType.DMA,
        ],
        compiler_params=pltpu.CompilerParams(needs_layout_passes=False),
    )
    def body(ids_hbm, table_hbm, pos_hbm, gamma_hbm, beta_hbm, out_hbm,
             ids_v, rows0, rows1, pos0, pos1, gamma_v, beta_v, pos_sh,
             p_sm, q_sm, sg0, sg1, sp0, sp1, so0, so1):
        rows = (rows0, rows1)
        pos = (pos0, pos1)
        sg = (sg0, sg1)
        sp = (sp0, sp1)
        so = (so0, so1)
        nc = 2
        wid = lax.axis_index("s") * nc + lax.axis_index("c")
        wbase = pl.multiple_of(wid * tok_per_w, CHUNK)

        pltpu.sync_copy(gamma_hbm, gamma_v)
        pltpu.sync_copy(beta_hbm, beta_v)
        pltpu.sync_copy(ids_hbm.at[pl.ds(wbase, tok_per_w)], ids_v)

        @pl.when(lax.axis_index("s") == 0)
        def _():
            pltpu.sync_copy(pos_hbm, pos_sh)

        plsc.subcore_barrier()

        def gather_desc(c, b):
            cb = pl.multiple_of(c * CHUNK, CHUNK)
            return pltpu.make_async_copy(
                table_hbm.at[ids_v.at[pl.ds(cb, CHUNK)]], rows[b], sg[b])

        def pos_desc(c, b):
            cb = pl.multiple_of(c * CHUNK, CHUNK)
            return pltpu.make_async_copy(
                pos_sh.at[pl.ds(cb, CHUNK)], pos[b], sp[b])

        def out_desc(c, b):
            ob = pl.multiple_of(wbase + c * CHUNK, CHUNK)
            return pltpu.make_async_copy(
                rows[b], out_hbm.at[pl.ds(ob, CHUNK)], so[b])

        def compute(b):
            rv = rows[b]
            pv = pos[b]

            def tok_body(t, tcarry):
                s = jnp.zeros((L,), jnp.float32)
                s2 = jnp.zeros((L,), jnp.float32)
                for i in range(NF):
                    sl = pl.ds(i * L, L)
                    v = rv[t, sl] + pv[t, sl]
                    rv[t, sl] = v
                    s = s + v
                    s2 = s2 + v * v
                tot = jnp.sum(s)
                tot2 = jnp.sum(s2)
                mean = tot * (1.0 / HIDDEN)
                var = tot2 * (1.0 / HIDDEN) - mean * mean
                rstd = _rsqrt_scalar(var + EPS)
                p_sm[t] = rstd
                q_sm[t] = -mean * rstd
                return tcarry

            lax.fori_loop(0, CHUNK, tok_body, 0)

            def feat_body(i, icarry):
                sl = pl.ds(pl.multiple_of(i * L, L), L)
                g = gamma_v[sl]
                be = beta_v[sl]
                for t in range(CHUNK):
                    x = rv[t, sl]
                    pvec = jnp.full((L,), p_sm[t], jnp.float32)
                    qvec = jnp.full((L,), q_sm[t], jnp.float32)
                    rv[t, sl] = (x * pvec + qvec) * g + be
                return icarry

            lax.fori_loop(0, NF, feat_body, 0)

        gather_desc(0, 0).start()
        pos_desc(0, 0).start()

        def pair_body(cc, carry):
            for b in (0, 1):
                c = cc * 2 + b
                nb = 1 - b
                gather_desc(c, b).wait()
                pos_desc(c, b).wait()

                @pl.when(jnp.logical_and(c >= 1, c < n_chunks - 1))
                def _():
                    out_desc(c - 1, nb).wait()

                @pl.when(c < n_chunks - 1)
                def _():
                    gather_desc(c + 1, nb).start()
                    pos_desc(c + 1, nb).start()

                compute(b)
                out_desc(c, b).start()
            return carry

        lax.fori_loop(0, n_chunks // 2, pair_body, 0)
        out_desc(n_chunks - 2, 0).wait()
        out_desc(n_chunks - 1, 1).wait()

    return body


def kernel(input_ids, word_emb, pos_emb, ln_gamma, ln_beta):
    b, s = input_ids.shape
    n_tokens = b * s
    info = plsc.get_sparse_core_info()
    n_workers = info.num_cores * info.num_subcores
    ids = input_ids.reshape(n_tokens).astype(jnp.int32)
    sc = _make_sc_kernel(n_tokens, n_workers, s)
    out = sc(ids, word_emb, pos_emb, ln_gamma, ln_beta)
    return out.reshape(b, s, HIDDEN)
